# Initial kernel scaffold; baseline (speedup 1.0000x reference)
import jax
import jax.numpy as jnp
from jax.experimental import pallas as pl


def kernel(pos, batch, normal, W1a, b1a, W1b, b1b, W2a, b2a, W2b, b2b, Wc, bc):
    raise NotImplementedError("write your pallas kernel here")


# speedup vs baseline: 2.8639x; 2.8639x over previous
import functools

import jax
import jax.numpy as jnp
from jax import lax
from jax.experimental import pallas as pl
from jax.experimental.pallas import tpu as pltpu
from jax.experimental.pallas import tpu_sc as plsc

_N = 10000
_K = 16
_G = 8
_BLK = 128
_NB = 79
_NP = _NB * _BLK
_E = _NP * _K

_HI = lax.Precision.HIGHEST


def _knn_body(pos_ref, posT_ref, brow_ref, bcol_ref, out_ref):
    i = pl.program_id(0)
    pb = pos_ref[...]
    pt = posT_ref[...]
    bb = brow_ref[...]
    bt = bcol_ref[...]
    sq_i = jnp.sum(pb * pb, axis=1, keepdims=True)
    sq_j = jnp.sum(pt * pt, axis=0, keepdims=True)
    g = jnp.dot(pb, pt, preferred_element_type=jnp.float32, precision=_HI)
    d = sq_i + sq_j - 2.0 * g
    col = lax.broadcasted_iota(jnp.int32, (1, _NP), 1)
    row = i * _BLK + lax.broadcasted_iota(jnp.int32, (_BLK, 1), 0)
    inf = jnp.float32(jnp.inf)
    d = jnp.where(bb == bt, d, inf)
    d = jnp.where(col == row, inf, d)
    big = jnp.int32(_NP)
    cols = []
    for _ in range(_K):
        m = jnp.min(d, axis=1, keepdims=True)
        cand = jnp.where(d == m, col, big)
        am = jnp.min(cand, axis=1, keepdims=True)
        cols.append(am)
        d = jnp.where(col == am, inf, d)
    out_ref[...] = jnp.concatenate(cols, axis=1)


def _knn(pos_pad, posT, brow, bcol):
    return pl.pallas_call(
        _knn_body,
        grid=(_NB,),
        in_specs=[
            pl.BlockSpec((_BLK, 8), lambda i: (i, 0)),
            pl.BlockSpec((8, _NP), lambda i: (0, 0)),
            pl.BlockSpec((_BLK, 1), lambda i: (i, 0)),
            pl.BlockSpec((1, _NP), lambda i: (0, 0)),
        ],
        out_specs=pl.BlockSpec((_BLK, _K), lambda i: (i, 0)),
        out_shape=jax.ShapeDtypeStruct((_NP, _K), jnp.int32),
    )(pos_pad, posT, brow, bcol)


def _gather_sc(table, idx):
    e = idx.shape[0]
    d = table.shape[1]
    info = plsc.get_sparse_core_info()
    nc, ns = info.num_cores, info.num_subcores
    nw = nc * ns
    per_w = e // nw
    n_chunks = 8
    ch = per_w // n_chunks

    mesh = plsc.VectorSubcoreMesh(core_axis_name="c", subcore_axis_name="s")

    @functools.partial(
        pl.kernel,
        mesh=mesh,
        compiler_params=pltpu.CompilerParams(use_tc_tiling_on_sc=False),
        out_type=jax.ShapeDtypeStruct((e, d), jnp.float32),
        scratch_types=[
            pltpu.VMEM((ch,), jnp.int32),
            pltpu.VMEM((ch, d), jnp.float32),
            pltpu.SemaphoreType.DMA,
        ],
    )
    def gk(table_hbm, idx_hbm, out_hbm, idx_v, rows_v, sem):
        wid = lax.axis_index("s") * nc + lax.axis_index("c")
        base = wid * per_w
        for c in range(n_chunks):
            off = base + c * ch
            pltpu.sync_copy(idx_hbm.at[pl.ds(off, ch)], idx_v)
            pltpu.async_copy(table_hbm.at[idx_v], rows_v, sem).wait()
            pltpu.sync_copy(rows_v, out_hbm.at[pl.ds(off, ch)])

    return gk(table, idx)


def _ppf_feats(sg, dg):
    px, py, pz = sg[:, 0:1], sg[:, 1:2], sg[:, 2:3]
    ax, ay, az = dg[:, 3:4], dg[:, 4:5], dg[:, 5:6]
    bx, by, bz = sg[:, 3:4], sg[:, 4:5], sg[:, 5:6]
    dx = px - dg[:, 0:1]
    dy = py - dg[:, 1:2]
    dz = pz - dg[:, 2:3]

    def ang(ux, uy, uz, vx, vy, vz):
        cx = uy * vz - uz * vy
        cy = uz * vx - ux * vz
        cz = ux * vy - uy * vx
        cn = jnp.sqrt(cx * cx + cy * cy + cz * cz)
        dp = ux * vx + uy * vy + uz * vz
        return jnp.arctan2(cn, dp)

    f1 = jnp.sqrt(dx * dx + dy * dy + dz * dz)
    f2 = ang(ax, ay, az, dx, dy, dz)
    f3 = ang(bx, by, bz, dx, dy, dz)
    f4 = ang(ax, ay, az, bx, by, bz)
    z = jnp.zeros_like(f1)
    return jnp.concatenate([f1, f2, f3, f4, z, z, z, z], axis=1)


def _conv1_body(sg_ref, dg_ref, wa_ref, ba_ref, wb_ref, bb_ref, out_ref):
    f = _ppf_feats(sg_ref[...], dg_ref[...])
    h = jnp.dot(f, wa_ref[...], preferred_element_type=jnp.float32,
                precision=_HI) + ba_ref[...]
    h = jnp.maximum(h, 0.0)
    m = jnp.dot(h, wb_ref[...], preferred_element_type=jnp.float32,
                precision=_HI) + bb_ref[...]
    r = jnp.max(m.reshape(_BLK, _K, 32), axis=1)
    out_ref[...] = jnp.maximum(r, 0.0)


def _conv2_body(sg_ref, dg_ref, hs_ref, wah_ref, waf_ref, ba_ref, wb_ref,
                bb_ref, out_ref):
    f = _ppf_feats(sg_ref[...], dg_ref[...])
    pre = (jnp.dot(hs_ref[...], wah_ref[...],
                   preferred_element_type=jnp.float32, precision=_HI)
           + jnp.dot(f, waf_ref[...],
                     preferred_element_type=jnp.float32, precision=_HI)
           + ba_ref[...])
    h = jnp.maximum(pre, 0.0)
    m = jnp.dot(h, wb_ref[...], preferred_element_type=jnp.float32,
                precision=_HI) + bb_ref[...]
    r = jnp.max(m.reshape(_BLK, _K, 32), axis=1)
    out_ref[...] = jnp.maximum(r, 0.0)


_EB = _BLK * _K


def _conv1(sg, dg, wa, ba, wb, bb):
    return pl.pallas_call(
        _conv1_body,
        grid=(_NB,),
        in_specs=[
            pl.BlockSpec((_EB, 16), lambda i: (i, 0)),
            pl.BlockSpec((_EB, 16), lambda i: (i, 0)),
            pl.BlockSpec((8, 32), lambda i: (0, 0)),
            pl.BlockSpec((1, 32), lambda i: (0, 0)),
            pl.BlockSpec((32, 32), lambda i: (0, 0)),
            pl.BlockSpec((1, 32), lambda i: (0, 0)),
        ],
        out_specs=pl.BlockSpec((_BLK, 32), lambda i: (i, 0)),
        out_shape=jax.ShapeDtypeStruct((_NP, 32), jnp.float32),
    )(sg, dg, wa, ba, wb, bb)


def _conv2(sg, dg, hs, wah, waf, ba, wb, bb):
    return pl.pallas_call(
        _conv2_body,
        grid=(_NB,),
        in_specs=[
            pl.BlockSpec((_EB, 16), lambda i: (i, 0)),
            pl.BlockSpec((_EB, 16), lambda i: (i, 0)),
            pl.BlockSpec((_EB, 32), lambda i: (i, 0)),
            pl.BlockSpec((32, 32), lambda i: (0, 0)),
            pl.BlockSpec((8, 32), lambda i: (0, 0)),
            pl.BlockSpec((1, 32), lambda i: (0, 0)),
            pl.BlockSpec((32, 32), lambda i: (0, 0)),
            pl.BlockSpec((1, 32), lambda i: (0, 0)),
        ],
        out_specs=pl.BlockSpec((_BLK, 32), lambda i: (i, 0)),
        out_shape=jax.ShapeDtypeStruct((_NP, 32), jnp.float32),
    )(sg, dg, hs, wah, waf, ba, wb, bb)


def _pool_body(h_ref, b_ref, wc_ref, bc_ref, out_ref):
    h = h_ref[...]
    b = b_ref[...]
    ninf = jnp.float32(-jnp.inf)
    segs = []
    for g in range(_G):
        hg = jnp.where(b == g, h, ninf)
        segs.append(jnp.max(hg, axis=0, keepdims=True))
    gmax = jnp.concatenate(segs, axis=0)
    out_ref[...] = jnp.dot(gmax, wc_ref[...], preferred_element_type=jnp.float32,
                           precision=_HI) + bc_ref[...]


def _pool(h, bcolumn, wc, bc):
    return pl.pallas_call(
        _pool_body,
        in_specs=[
            pl.BlockSpec((_N, 32), lambda: (0, 0)),
            pl.BlockSpec((_N, 1), lambda: (0, 0)),
            pl.BlockSpec((32, 40), lambda: (0, 0)),
            pl.BlockSpec((1, 40), lambda: (0, 0)),
        ],
        out_specs=pl.BlockSpec((_G, 40), lambda: (0, 0)),
        out_shape=jax.ShapeDtypeStruct((_G, 40), jnp.float32),
    )(h, bcolumn, wc, bc)


def kernel(pos, batch, normal, W1a, b1a, W1b, b1b, W2a, b2a, W2b, b2b, Wc, bc):
    batch = batch.astype(jnp.int32)
    pad = _NP - _N

    pos_pad = jnp.pad(pos, ((0, pad), (0, 5)))
    posT = pos_pad.T
    batch_pad = jnp.pad(batch, (0, pad), constant_values=-1)
    brow = batch_pad.reshape(_NP, 1)
    bcol = batch_pad.reshape(1, _NP)

    idx = _knn(pos_pad, posT, brow, bcol)

    src = idx.reshape(-1)
    dst = jnp.repeat(jnp.arange(_NP, dtype=jnp.int32), _K)

    geo = jnp.pad(jnp.concatenate([pos, normal], axis=1), ((0, pad), (0, 10)))
    sg = _gather_sc(geo, src)
    dg = _gather_sc(geo, dst)

    w1a = jnp.pad(W1a, ((0, 4), (0, 0)))
    h1 = _conv1(sg, dg, w1a, b1a.reshape(1, 32), W1b, b1b.reshape(1, 32))

    hs = _gather_sc(h1, src)
    w2af = jnp.pad(W2a[32:36], ((0, 4), (0, 0)))
    h2 = _conv2(sg, dg, hs, W2a[:32], w2af, b2a.reshape(1, 32),
                W2b, b2b.reshape(1, 32))

    return _pool(h2[:_N], batch.reshape(_N, 1), Wc, bc.reshape(1, 40))


# speedup vs baseline: 3.3318x; 1.1634x over previous
import functools

import jax
import jax.numpy as jnp
from jax import lax
from jax.experimental import pallas as pl
from jax.experimental.pallas import tpu as pltpu
from jax.experimental.pallas import tpu_sc as plsc

_N = 10000
_K = 16
_G = 8
_BLK = 128
_NB = 80
_NP = _NB * _BLK
_E = _NP * _K
_CW = 512

_HI = lax.Precision.HIGHEST


def _knn_body(lo_ref, hi_ref, pos_ref, posT_ref, brow_ref, bcol_ref, out_ref):
    i = pl.program_id(0)
    pb = pos_ref[...]
    bb = brow_ref[...]
    sq_i = jnp.sum(pb * pb, axis=1, keepdims=True)
    row = i * _BLK + lax.broadcasted_iota(jnp.int32, (_BLK, 1), 0)
    inf = jnp.float32(jnp.inf)
    bigi = jnp.int32(_NP)

    rv0 = jnp.full((_BLK, _K), inf, jnp.float32)
    ri0 = jnp.full((_BLK, _K), bigi, jnp.int32)

    def chunk(j, carry):
        rv, ri = carry
        c0 = j * _CW
        pt = posT_ref[:, pl.ds(c0, _CW)]
        bt = bcol_ref[:, pl.ds(c0, _CW)]
        sq_j = jnp.sum(pt * pt, axis=0, keepdims=True)
        g = jnp.dot(pb, pt, preferred_element_type=jnp.float32, precision=_HI)
        d = sq_i + sq_j - 2.0 * g
        colc = c0 + lax.broadcasted_iota(jnp.int32, (1, _CW), 1)
        d = jnp.where(bb == bt, d, inf)
        d = jnp.where(colc == row, inf, d)
        vals = jnp.concatenate([rv, d], axis=1)
        idxs = jnp.concatenate(
            [ri, jnp.broadcast_to(colc, (_BLK, _CW))], axis=1)
        nv, ni = [], []
        for _ in range(_K):
            m = jnp.min(vals, axis=1, keepdims=True)
            cand = jnp.where(vals == m, idxs, bigi)
            am = jnp.min(cand, axis=1, keepdims=True)
            nv.append(m)
            ni.append(am)
            vals = jnp.where(idxs == am, inf, vals)
        return jnp.concatenate(nv, axis=1), jnp.concatenate(ni, axis=1)

    _, ri = lax.fori_loop(lo_ref[i], hi_ref[i], chunk, (rv0, ri0))
    out_ref[...] = jnp.minimum(ri, bigi - 1)


def _knn(lo, hi, pos_pad, posT, brow, bcol):
    return pl.pallas_call(
        _knn_body,
        grid=(_NB,),
        in_specs=[
            pl.BlockSpec(memory_space=pltpu.SMEM),
            pl.BlockSpec(memory_space=pltpu.SMEM),
            pl.BlockSpec((_BLK, 8), lambda i: (i, 0)),
            pl.BlockSpec((8, _NP), lambda i: (0, 0)),
            pl.BlockSpec((_BLK, 1), lambda i: (i, 0)),
            pl.BlockSpec((1, _NP), lambda i: (0, 0)),
        ],
        out_specs=pl.BlockSpec((_BLK, _K), lambda i: (i, 0)),
        out_shape=jax.ShapeDtypeStruct((_NP, _K), jnp.int32),
    )(lo, hi, pos_pad, posT, brow, bcol)


def _gather_sc(table, idx):
    e = idx.shape[0]
    d = table.shape[1]
    info = plsc.get_sparse_core_info()
    nc, ns = info.num_cores, info.num_subcores
    nw = nc * ns
    per_w = e // nw
    n_chunks = 8
    ch = per_w // n_chunks

    mesh = plsc.VectorSubcoreMesh(core_axis_name="c", subcore_axis_name="s")

    @functools.partial(
        pl.kernel,
        mesh=mesh,
        compiler_params=pltpu.CompilerParams(use_tc_tiling_on_sc=False),
        out_type=jax.ShapeDtypeStruct((e, d), jnp.float32),
        scratch_types=[
            pltpu.VMEM((ch,), jnp.int32),
            pltpu.VMEM((ch, d), jnp.float32),
            pltpu.SemaphoreType.DMA,
        ],
    )
    def gk(table_hbm, idx_hbm, out_hbm, idx_v, rows_v, sem):
        wid = lax.axis_index("s") * nc + lax.axis_index("c")
        base = wid * per_w
        for c in range(n_chunks):
            off = base + c * ch
            pltpu.sync_copy(idx_hbm.at[pl.ds(off, ch)], idx_v)
            pltpu.async_copy(table_hbm.at[idx_v], rows_v, sem).wait()
            pltpu.sync_copy(rows_v, out_hbm.at[pl.ds(off, ch)])

    return gk(table, idx)


def _ppf_feats(sg, dg):
    px, py, pz = sg[:, 0:1], sg[:, 1:2], sg[:, 2:3]
    ax, ay, az = dg[:, 3:4], dg[:, 4:5], dg[:, 5:6]
    bx, by, bz = sg[:, 3:4], sg[:, 4:5], sg[:, 5:6]
    dx = px - dg[:, 0:1]
    dy = py - dg[:, 1:2]
    dz = pz - dg[:, 2:3]

    def ang(ux, uy, uz, vx, vy, vz):
        cx = uy * vz - uz * vy
        cy = uz * vx - ux * vz
        cz = ux * vy - uy * vx
        cn = jnp.sqrt(cx * cx + cy * cy + cz * cz)
        dp = ux * vx + uy * vy + uz * vz
        return jnp.arctan2(cn, dp)

    f1 = jnp.sqrt(dx * dx + dy * dy + dz * dz)
    f2 = ang(ax, ay, az, dx, dy, dz)
    f3 = ang(bx, by, bz, dx, dy, dz)
    f4 = ang(ax, ay, az, bx, by, bz)
    z = jnp.zeros_like(f1)
    return jnp.concatenate([f1, f2, f3, f4, z, z, z, z], axis=1)


def _conv1_body(sg_ref, dg_ref, wa_ref, ba_ref, wb_ref, bb_ref, out_ref):
    f = _ppf_feats(sg_ref[...], dg_ref[...])
    h = jnp.dot(f, wa_ref[...], preferred_element_type=jnp.float32,
                precision=_HI) + ba_ref[...]
    h = jnp.maximum(h, 0.0)
    m = jnp.dot(h, wb_ref[...], preferred_element_type=jnp.float32,
                precision=_HI) + bb_ref[...]
    r = jnp.max(m.reshape(_BLK, _K, 32), axis=1)
    out_ref[...] = jnp.maximum(r, 0.0)


def _conv2_body(sg_ref, dg_ref, hs_ref, wah_ref, waf_ref, ba_ref, wb_ref,
                bb_ref, out_ref):
    f = _ppf_feats(sg_ref[...], dg_ref[...])
    pre = (jnp.dot(hs_ref[...], wah_ref[...],
                   preferred_element_type=jnp.float32, precision=_HI)
           + jnp.dot(f, waf_ref[...],
                     preferred_element_type=jnp.float32, precision=_HI)
           + ba_ref[...])
    h = jnp.maximum(pre, 0.0)
    m = jnp.dot(h, wb_ref[...], preferred_element_type=jnp.float32,
                precision=_HI) + bb_ref[...]
    r = jnp.max(m.reshape(_BLK, _K, 32), axis=1)
    out_ref[...] = jnp.maximum(r, 0.0)


_EB = _BLK * _K


def _conv1(sg, dg, wa, ba, wb, bb):
    return pl.pallas_call(
        _conv1_body,
        grid=(_NB,),
        in_specs=[
            pl.BlockSpec((_EB, 16), lambda i: (i, 0)),
            pl.BlockSpec((_EB, 16), lambda i: (i, 0)),
            pl.BlockSpec((8, 32), lambda i: (0, 0)),
            pl.BlockSpec((1, 32), lambda i: (0, 0)),
            pl.BlockSpec((32, 32), lambda i: (0, 0)),
            pl.BlockSpec((1, 32), lambda i: (0, 0)),
        ],
        out_specs=pl.BlockSpec((_BLK, 32), lambda i: (i, 0)),
        out_shape=jax.ShapeDtypeStruct((_NP, 32), jnp.float32),
    )(sg, dg, wa, ba, wb, bb)


def _conv2(sg, dg, hs, wah, waf, ba, wb, bb):
    return pl.pallas_call(
        _conv2_body,
        grid=(_NB,),
        in_specs=[
            pl.BlockSpec((_EB, 16), lambda i: (i, 0)),
            pl.BlockSpec((_EB, 16), lambda i: (i, 0)),
            pl.BlockSpec((_EB, 32), lambda i: (i, 0)),
            pl.BlockSpec((32, 32), lambda i: (0, 0)),
            pl.BlockSpec((8, 32), lambda i: (0, 0)),
            pl.BlockSpec((1, 32), lambda i: (0, 0)),
            pl.BlockSpec((32, 32), lambda i: (0, 0)),
            pl.BlockSpec((1, 32), lambda i: (0, 0)),
        ],
        out_specs=pl.BlockSpec((_BLK, 32), lambda i: (i, 0)),
        out_shape=jax.ShapeDtypeStruct((_NP, 32), jnp.float32),
    )(sg, dg, hs, wah, waf, ba, wb, bb)


def _pool_body(h_ref, b_ref, wc_ref, bc_ref, out_ref):
    h = h_ref[...]
    b = b_ref[...]
    ninf = jnp.float32(-jnp.inf)
    segs = []
    for g in range(_G):
        hg = jnp.where(b == g, h, ninf)
        segs.append(jnp.max(hg, axis=0, keepdims=True))
    gmax = jnp.concatenate(segs, axis=0)
    out_ref[...] = jnp.dot(gmax, wc_ref[...], preferred_element_type=jnp.float32,
                           precision=_HI) + bc_ref[...]


def _pool(h, bcolumn, wc, bc):
    return pl.pallas_call(
        _pool_body,
        in_specs=[
            pl.BlockSpec((_N, 32), lambda: (0, 0)),
            pl.BlockSpec((_N, 1), lambda: (0, 0)),
            pl.BlockSpec((32, 40), lambda: (0, 0)),
            pl.BlockSpec((1, 40), lambda: (0, 0)),
        ],
        out_specs=pl.BlockSpec((_G, 40), lambda: (0, 0)),
        out_shape=jax.ShapeDtypeStruct((_G, 40), jnp.float32),
    )(h, bcolumn, wc, bc)


def kernel(pos, batch, normal, W1a, b1a, W1b, b1b, W2a, b2a, W2b, b2b, Wc, bc):
    batch = batch.astype(jnp.int32)
    pad = _NP - _N

    pos_pad = jnp.pad(pos, ((0, pad), (0, 5)))
    posT = pos_pad.T
    batch_pad = jnp.pad(batch, (0, pad), constant_values=-1)
    brow = batch_pad.reshape(_NP, 1)
    bcol = batch_pad.reshape(1, _NP)

    gids = jnp.arange(_G, dtype=jnp.int32)
    starts = jnp.searchsorted(batch, gids, side="left").astype(jnp.int32)
    ends = jnp.searchsorted(batch, gids, side="right").astype(jnp.int32)
    bi = jnp.arange(_NB, dtype=jnp.int32) * _BLK
    first = batch_pad[bi]
    last = batch_pad[bi + _BLK - 1]
    lo_col = jnp.where(first >= 0, starts[first], _N)
    hi_col = jnp.where(last >= 0, ends[last], _NP)
    hi_col = jnp.where(last == -1, _NP, hi_col)
    lo = (lo_col // _CW).astype(jnp.int32)
    hi = ((hi_col + _CW - 1) // _CW).astype(jnp.int32)

    idx = _knn(lo, hi, pos_pad, posT, brow, bcol)

    src = idx.reshape(-1)
    dst = jnp.repeat(jnp.arange(_NP, dtype=jnp.int32), _K)

    geo = jnp.pad(jnp.concatenate([pos, normal], axis=1), ((0, pad), (0, 10)))
    sg = _gather_sc(geo, src)
    dg = _gather_sc(geo, dst)

    w1a = jnp.pad(W1a, ((0, 4), (0, 0)))
    h1 = _conv1(sg, dg, w1a, b1a.reshape(1, 32), W1b, b1b.reshape(1, 32))

    hs = _gather_sc(h1, src)
    w2af = jnp.pad(W2a[32:36], ((0, 4), (0, 0)))
    h2 = _conv2(sg, dg, hs, W2a[:32], w2af, b2a.reshape(1, 32),
                W2b, b2b.reshape(1, 32))

    return _pool(h2[:_N], batch.reshape(_N, 1), Wc, bc.reshape(1, 40))


# speedup vs baseline: 3.7919x; 1.1381x over previous
import functools

import jax
import jax.numpy as jnp
from jax import lax
from jax.experimental import pallas as pl
from jax.experimental.pallas import tpu as pltpu
from jax.experimental.pallas import tpu_sc as plsc

_N = 10000
_K = 16
_G = 8
_BLK = 256
_NB = 40
_NP = _NB * _BLK
_E = _NP * _K
_CW = 512

_HI = lax.Precision.HIGHEST


def _knn_body(lo_ref, hi_ref, pos_ref, posT_ref, brow_ref, bcol_ref, out_ref):
    i = pl.program_id(0)
    pb = pos_ref[...]
    bb = brow_ref[...]
    sq_i = jnp.sum(pb * pb, axis=1, keepdims=True)
    row = i * _BLK + lax.broadcasted_iota(jnp.int32, (_BLK, 1), 0)
    inf = jnp.float32(jnp.inf)
    bigi = jnp.int32(_NP)

    rv0 = jnp.full((_BLK, _K), inf, jnp.float32)
    ri0 = jnp.full((_BLK, _K), bigi, jnp.int32)

    def chunk(j, carry):
        rv, ri = carry
        c0 = j * _CW
        pt = posT_ref[:, pl.ds(c0, _CW)]
        bt = bcol_ref[:, pl.ds(c0, _CW)]
        sq_j = jnp.sum(pt * pt, axis=0, keepdims=True)
        g = jnp.dot(pb, pt, preferred_element_type=jnp.float32, precision=_HI)
        d = sq_i + sq_j - 2.0 * g
        colc = c0 + lax.broadcasted_iota(jnp.int32, (1, _CW), 1)
        d = jnp.where(bb == bt, d, inf)
        d = jnp.where(colc == row, inf, d)
        vals = jnp.concatenate([rv, d], axis=1)
        idxs = jnp.concatenate(
            [ri, jnp.broadcast_to(colc, (_BLK, _CW))], axis=1)
        nv, ni = [], []
        for _ in range(_K):
            m = jnp.min(vals, axis=1, keepdims=True)
            cand = jnp.where(vals == m, idxs, bigi)
            am = jnp.min(cand, axis=1, keepdims=True)
            nv.append(m)
            ni.append(am)
            vals = jnp.where(idxs == am, inf, vals)
        return jnp.concatenate(nv, axis=1), jnp.concatenate(ni, axis=1)

    _, ri = lax.fori_loop(lo_ref[i], hi_ref[i], chunk, (rv0, ri0))
    out_ref[...] = jnp.minimum(ri, bigi - 1)


def _knn(lo, hi, pos_pad, posT, brow, bcol):
    return pl.pallas_call(
        _knn_body,
        grid=(_NB,),
        in_specs=[
            pl.BlockSpec(memory_space=pltpu.SMEM),
            pl.BlockSpec(memory_space=pltpu.SMEM),
            pl.BlockSpec((_BLK, 8), lambda i: (i, 0)),
            pl.BlockSpec((8, _NP), lambda i: (0, 0)),
            pl.BlockSpec((_BLK, 1), lambda i: (i, 0)),
            pl.BlockSpec((1, _NP), lambda i: (0, 0)),
        ],
        out_specs=pl.BlockSpec((_BLK, _K), lambda i: (i, 0)),
        out_shape=jax.ShapeDtypeStruct((_NP, _K), jnp.int32),
    )(lo, hi, pos_pad, posT, brow, bcol)


def _gather_sc(table, idx):
    e = idx.shape[0]
    d = table.shape[1]
    info = plsc.get_sparse_core_info()
    nc, ns = info.num_cores, info.num_subcores
    nw = nc * ns
    per_w = e // nw
    n_chunks = 8
    ch = per_w // n_chunks

    mesh = plsc.VectorSubcoreMesh(core_axis_name="c", subcore_axis_name="s")

    @functools.partial(
        pl.kernel,
        mesh=mesh,
        compiler_params=pltpu.CompilerParams(use_tc_tiling_on_sc=False),
        out_type=jax.ShapeDtypeStruct((e, d), jnp.float32),
        scratch_types=[
            pltpu.VMEM((ch,), jnp.int32),
            pltpu.VMEM((ch, d), jnp.float32),
            pltpu.SemaphoreType.DMA,
        ],
    )
    def gk(table_hbm, idx_hbm, out_hbm, idx_v, rows_v, sem):
        wid = lax.axis_index("s") * nc + lax.axis_index("c")
        base = wid * per_w
        for c in range(n_chunks):
            off = base + c * ch
            pltpu.sync_copy(idx_hbm.at[pl.ds(off, ch)], idx_v)
            pltpu.async_copy(table_hbm.at[idx_v], rows_v, sem).wait()
            pltpu.sync_copy(rows_v, out_hbm.at[pl.ds(off, ch)])

    return gk(table, idx)


def _ppf_feats(sg, dg):
    px, py, pz = sg[:, 0:1], sg[:, 1:2], sg[:, 2:3]
    ax, ay, az = dg[:, 3:4], dg[:, 4:5], dg[:, 5:6]
    bx, by, bz = sg[:, 3:4], sg[:, 4:5], sg[:, 5:6]
    dx = px - dg[:, 0:1]
    dy = py - dg[:, 1:2]
    dz = pz - dg[:, 2:3]

    def ang(ux, uy, uz, vx, vy, vz):
        cx = uy * vz - uz * vy
        cy = uz * vx - ux * vz
        cz = ux * vy - uy * vx
        cn = jnp.sqrt(cx * cx + cy * cy + cz * cz)
        dp = ux * vx + uy * vy + uz * vz
        return jnp.arctan2(cn, dp)

    f1 = jnp.sqrt(dx * dx + dy * dy + dz * dz)
    f2 = ang(ax, ay, az, dx, dy, dz)
    f3 = ang(bx, by, bz, dx, dy, dz)
    f4 = ang(ax, ay, az, bx, by, bz)
    z = jnp.zeros_like(f1)
    return jnp.concatenate([f1, f2, f3, f4, z, z, z, z], axis=1)


def _conv1_body(sg_ref, dg_ref, wa_ref, ba_ref, wb_ref, bb_ref, out_ref):
    f = _ppf_feats(sg_ref[...], dg_ref[...])
    h = jnp.dot(f, wa_ref[...], preferred_element_type=jnp.float32,
                precision=_HI) + ba_ref[...]
    h = jnp.maximum(h, 0.0)
    m = jnp.dot(h, wb_ref[...], preferred_element_type=jnp.float32,
                precision=_HI) + bb_ref[...]
    r = jnp.max(m.reshape(_BLK, _K, 32), axis=1)
    out_ref[...] = jnp.maximum(r, 0.0)


def _conv2_body(sg_ref, dg_ref, hs_ref, wah_ref, waf_ref, ba_ref, wb_ref,
                bb_ref, out_ref):
    f = _ppf_feats(sg_ref[...], dg_ref[...])
    pre = (jnp.dot(hs_ref[...], wah_ref[...],
                   preferred_element_type=jnp.float32, precision=_HI)
           + jnp.dot(f, waf_ref[...],
                     preferred_element_type=jnp.float32, precision=_HI)
           + ba_ref[...])
    h = jnp.maximum(pre, 0.0)
    m = jnp.dot(h, wb_ref[...], preferred_element_type=jnp.float32,
                precision=_HI) + bb_ref[...]
    r = jnp.max(m.reshape(_BLK, _K, 32), axis=1)
    out_ref[...] = jnp.maximum(r, 0.0)


_EB = _BLK * _K


def _conv1(sg, dg, wa, ba, wb, bb):
    return pl.pallas_call(
        _conv1_body,
        grid=(_NB,),
        in_specs=[
            pl.BlockSpec((_EB, 16), lambda i: (i, 0)),
            pl.BlockSpec((_EB, 16), lambda i: (i, 0)),
            pl.BlockSpec((8, 32), lambda i: (0, 0)),
            pl.BlockSpec((1, 32), lambda i: (0, 0)),
            pl.BlockSpec((32, 32), lambda i: (0, 0)),
            pl.BlockSpec((1, 32), lambda i: (0, 0)),
        ],
        out_specs=pl.BlockSpec((_BLK, 32), lambda i: (i, 0)),
        out_shape=jax.ShapeDtypeStruct((_NP, 32), jnp.float32),
    )(sg, dg, wa, ba, wb, bb)


def _conv2(sg, dg, hs, wah, waf, ba, wb, bb):
    return pl.pallas_call(
        _conv2_body,
        grid=(_NB,),
        in_specs=[
            pl.BlockSpec((_EB, 16), lambda i: (i, 0)),
            pl.BlockSpec((_EB, 16), lambda i: (i, 0)),
            pl.BlockSpec((_EB, 32), lambda i: (i, 0)),
            pl.BlockSpec((32, 32), lambda i: (0, 0)),
            pl.BlockSpec((8, 32), lambda i: (0, 0)),
            pl.BlockSpec((1, 32), lambda i: (0, 0)),
            pl.BlockSpec((32, 32), lambda i: (0, 0)),
            pl.BlockSpec((1, 32), lambda i: (0, 0)),
        ],
        out_specs=pl.BlockSpec((_BLK, 32), lambda i: (i, 0)),
        out_shape=jax.ShapeDtypeStruct((_NP, 32), jnp.float32),
    )(sg, dg, hs, wah, waf, ba, wb, bb)


def _pool_body(h_ref, b_ref, wc_ref, bc_ref, out_ref):
    h = h_ref[...]
    b = b_ref[...]
    ninf = jnp.float32(-jnp.inf)
    segs = []
    for g in range(_G):
        hg = jnp.where(b == g, h, ninf)
        segs.append(jnp.max(hg, axis=0, keepdims=True))
    gmax = jnp.concatenate(segs, axis=0)
    out_ref[...] = jnp.dot(gmax, wc_ref[...], preferred_element_type=jnp.float32,
                           precision=_HI) + bc_ref[...]


def _pool(h, bcolumn, wc, bc):
    return pl.pallas_call(
        _pool_body,
        in_specs=[
            pl.BlockSpec((_N, 32), lambda: (0, 0)),
            pl.BlockSpec((_N, 1), lambda: (0, 0)),
            pl.BlockSpec((32, 40), lambda: (0, 0)),
            pl.BlockSpec((1, 40), lambda: (0, 0)),
        ],
        out_specs=pl.BlockSpec((_G, 40), lambda: (0, 0)),
        out_shape=jax.ShapeDtypeStruct((_G, 40), jnp.float32),
    )(h, bcolumn, wc, bc)


def kernel(pos, batch, normal, W1a, b1a, W1b, b1b, W2a, b2a, W2b, b2b, Wc, bc):
    batch = batch.astype(jnp.int32)
    pad = _NP - _N

    pos_pad = jnp.pad(pos, ((0, pad), (0, 5)))
    posT = pos_pad.T
    batch_pad = jnp.pad(batch, (0, pad), constant_values=-1)
    brow = batch_pad.reshape(_NP, 1)
    bcol = batch_pad.reshape(1, _NP)

    gids = jnp.arange(_G, dtype=jnp.int32)
    starts = jnp.searchsorted(batch, gids, side="left").astype(jnp.int32)
    ends = jnp.searchsorted(batch, gids, side="right").astype(jnp.int32)
    bi = jnp.arange(_NB, dtype=jnp.int32) * _BLK
    first = batch_pad[bi]
    last = batch_pad[bi + _BLK - 1]
    lo_col = jnp.where(first >= 0, starts[first], _N)
    hi_col = jnp.where(last >= 0, ends[last], _NP)
    hi_col = jnp.where(last == -1, _NP, hi_col)
    lo = (lo_col // _CW).astype(jnp.int32)
    hi = ((hi_col + _CW - 1) // _CW).astype(jnp.int32)

    idx = _knn(lo, hi, pos_pad, posT, brow, bcol)

    src = idx.reshape(-1)
    dst = jnp.repeat(jnp.arange(_NP, dtype=jnp.int32), _K)

    geo = jnp.pad(jnp.concatenate([pos, normal], axis=1), ((0, pad), (0, 10)))
    sg = _gather_sc(geo, src)
    dg = _gather_sc(geo, dst)

    w1a = jnp.pad(W1a, ((0, 4), (0, 0)))
    h1 = _conv1(sg, dg, w1a, b1a.reshape(1, 32), W1b, b1b.reshape(1, 32))

    hs = _gather_sc(h1, src)
    w2af = jnp.pad(W2a[32:36], ((0, 4), (0, 0)))
    h2 = _conv2(sg, dg, hs, W2a[:32], w2af, b2a.reshape(1, 32),
                W2b, b2b.reshape(1, 32))

    return _pool(h2[:_N], batch.reshape(_N, 1), Wc, bc.reshape(1, 40))
